# baseline (speedup 1.0000x reference)
import functools

import jax
import jax.numpy as jnp
import numpy as np
from jax import lax
from jax.experimental import pallas as pl
from jax.experimental.pallas import tpu as pltpu
from jax.experimental.pallas import tpu_sc as plsc

N = 10000
E = 320000
D = 128
NPAD = 10240
NC, NS, L = 2, 16, 16
NW = NC * NS

SCK = 128
SNCH = 80
EWP = SCK * SNCH
DCK = 128
DNCH = 160
DEGPAD = 10496

_HIGH = jax.lax.Precision.DEFAULT


def _mesh():
    return plsc.VectorSubcoreMesh(
        core_axis_name="c", subcore_axis_name="s", num_cores=NC, num_subcores=NS
    )


def _fast_rsqrt16(v):
    i = lax.bitcast_convert_type(v, jnp.int32)
    i = jnp.int32(0x5F3759DF) - (i >> 1)
    y = lax.bitcast_convert_type(i, jnp.float32)
    for _ in range(3):
        y = y * (1.5 - 0.5 * v * y * y)
    return y


def _degree_body(sidx_hbm, didx_hbm, norms_hbm, idx_v, ones_v, init_v, buf_v,
                 sem, deg_sh):
    c = lax.axis_index("c")
    s = lax.axis_index("s")

    @pl.when(c == 0)
    def _():
        pltpu.sync_copy(sidx_hbm.at[s], idx_v)

    @pl.when(c == 1)
    def _():
        pltpu.sync_copy(didx_hbm.at[s], idx_v)

    def fill_ones(i, _):
        ones_v[pl.ds(i * L, L)] = jnp.full((L,), 1.0, jnp.float32)
        return _

    lax.fori_loop(0, DCK // L, fill_ones, None)

    def fill_init(i, _):
        init_v[pl.ds(i * L, L)] = jnp.full((L,), 1.0, jnp.float32)
        return _

    lax.fori_loop(0, (DEGPAD // NS) // L, fill_init, None)
    pltpu.sync_copy(init_v, deg_sh.at[pl.ds(s * (DEGPAD // NS), DEGPAD // NS)])
    plsc.subcore_barrier()

    def chunk(j, _):
        for k in range(8):
            pltpu.async_copy(ones_v, deg_sh.at[idx_v.at[j * 8 + k]], sem,
                             add=True)
        for k in range(8):
            pltpu.make_async_copy(
                ones_v, deg_sh.at[idx_v.at[j * 8 + k]], sem).wait()
        return _

    lax.fori_loop(0, DNCH // 8, chunk, None)
    plsc.subcore_barrier()

    nper = NPAD // NS
    pltpu.sync_copy(deg_sh.at[pl.ds(s * nper, nper)], buf_v)

    def rsq(i, _):
        buf_v[pl.ds(i * L, L)] = _fast_rsqrt16(buf_v[pl.ds(i * L, L)])
        return _

    lax.fori_loop(0, nper // L, rsq, None)
    pltpu.sync_copy(buf_v, norms_hbm.at[c, pl.ds(s * nper, nper)])


def _degrees(sidx_r, didx_r):
    kern = pl.kernel(
        _degree_body,
        out_type=jax.ShapeDtypeStruct((NC, NPAD), jnp.float32),
        mesh=_mesh(),
        scratch_types=[
            pltpu.VMEM((DNCH, DCK), jnp.int32),
            pltpu.VMEM((DCK,), jnp.float32),
            pltpu.VMEM((DEGPAD // NS,), jnp.float32),
            pltpu.VMEM((NPAD // NS,), jnp.float32),
            pltpu.SemaphoreType.DMA,
            pltpu.VMEM_SHARED((DEGPAD,), jnp.float32),
        ],
    )
    return kern(sidx_r, didx_r)


def _scatter_body(h_hbm, sidx_hbm, didx_hbm, out_hbm,
                  sidx_v, didx_v, rows_v, sem_a, sem_b, sem_c, sem_d, agg_sh):
    c = lax.axis_index("c")
    s = lax.axis_index("s")
    wid = c * NS + s
    nper = NPAD // NS

    def zrow(i, _):
        for k in range(D // L):
            rows_v[0, i, pl.ds(k * L, L)] = jnp.zeros((L,), jnp.float32)
        return _

    lax.fori_loop(0, SCK, zrow, None)
    for k in range(nper // SCK):
        pltpu.sync_copy(rows_v.at[0], agg_sh.at[pl.ds(s * nper + k * SCK, SCK)])

    plsc.subcore_barrier()

    rows0, rows1 = rows_v.at[0], rows_v.at[1]
    half = SNCH // 2
    for p in range(2):
        pltpu.sync_copy(sidx_hbm.at[wid, pl.ds(p * half, half)], sidx_v)
        pltpu.sync_copy(didx_hbm.at[wid, pl.ds(p * half, half)], didx_v)
        pltpu.async_copy(h_hbm.at[sidx_v.at[0]], rows0, sem_a)
        pltpu.async_copy(h_hbm.at[sidx_v.at[1]], rows1, sem_b)

        def body(i, _):
            g = 2 * i
            pltpu.make_async_copy(h_hbm.at[sidx_v.at[g]], rows0, sem_a).wait()
            s0 = pltpu.async_copy(rows0, agg_sh.at[didx_v.at[g]], sem_c,
                                  add=True)
            pltpu.make_async_copy(h_hbm.at[sidx_v.at[g + 1]], rows1, sem_b).wait()
            s1 = pltpu.async_copy(rows1, agg_sh.at[didx_v.at[g + 1]], sem_d,
                                  add=True)
            s0.wait()

            @pl.when(i < half // 2 - 1)
            def _start_next0():
                pltpu.async_copy(h_hbm.at[sidx_v.at[g + 2]], rows0, sem_a)

            s1.wait()

            @pl.when(i < half // 2 - 1)
            def _start_next1():
                pltpu.async_copy(h_hbm.at[sidx_v.at[g + 3]], rows1, sem_b)

            return _

        lax.fori_loop(0, half // 2, body, None)
    plsc.subcore_barrier()

    pltpu.sync_copy(agg_sh.at[pl.ds(s * nper, nper)],
                    out_hbm.at[c, pl.ds(s * nper, nper)])


def _edge_scatter(h, sidx, didx):
    kern = pl.kernel(
        _scatter_body,
        out_type=jax.ShapeDtypeStruct((NC, NPAD, D), jnp.float32),
        mesh=_mesh(),
        scratch_types=[
            pltpu.VMEM((SNCH // 2, SCK), jnp.int32),
            pltpu.VMEM((SNCH // 2, SCK), jnp.int32),
            pltpu.VMEM((2, SCK, D), jnp.float32),
            pltpu.SemaphoreType.DMA,
            pltpu.SemaphoreType.DMA,
            pltpu.SemaphoreType.DMA,
            pltpu.SemaphoreType.DMA,
            pltpu.VMEM_SHARED((NPAD, D), jnp.float32),
        ],
    )
    return kern(h, sidx, didx)


BR = 2048


def _mm1_body(x_ref, ns_ref, w_ref, o_ref):
    o_ref[...] = jnp.dot(x_ref[...] * ns_ref[...], w_ref[...],
                         preferred_element_type=jnp.float32, precision=_HIGH)


def _mm1(xp, nsrc, W1):
    return pl.pallas_call(
        _mm1_body,
        grid=(NPAD // BR,),
        in_specs=[
            pl.BlockSpec((BR, D), lambda j: (j, 0)),
            pl.BlockSpec((BR, 1), lambda j: (j, 0)),
            pl.BlockSpec((D, D), lambda j: (0, 0)),
        ],
        out_specs=pl.BlockSpec((BR, D), lambda j: (j, 0)),
        out_shape=jax.ShapeDtypeStruct((NPAD, D), jnp.float32),
    )(xp, nsrc, W1)


def _combine_body(agg_ref, hin_ref, ns_ref, nd_ref, b_ref, w_ref, o_ref):
    a = agg_ref[0] + agg_ref[1] + hin_ref[...]
    h = jax.nn.relu(a * nd_ref[...] + b_ref[...])
    o_ref[...] = jnp.dot(h * ns_ref[...], w_ref[...],
                         preferred_element_type=jnp.float32, precision=_HIGH)


def _combine(agg, hin, nsrc, ndst, b, W):
    return pl.pallas_call(
        _combine_body,
        grid=(NPAD // BR,),
        in_specs=[
            pl.BlockSpec((NC, BR, D), lambda j: (0, j, 0)),
            pl.BlockSpec((BR, D), lambda j: (j, 0)),
            pl.BlockSpec((BR, 1), lambda j: (j, 0)),
            pl.BlockSpec((BR, 1), lambda j: (j, 0)),
            pl.BlockSpec((1, D), lambda j: (0, 0)),
            pl.BlockSpec((D, D), lambda j: (0, 0)),
        ],
        out_specs=pl.BlockSpec((BR, D), lambda j: (j, 0)),
        out_shape=jax.ShapeDtypeStruct((NPAD, D), jnp.float32),
    )(agg, hin, nsrc, ndst, b, W)


def _final_body(agg_ref, hin_ref, nd_ref, b_ref,
                wf1_ref, bf1_ref, wf2_ref, bf2_ref, wf3_ref, bf3_ref,
                o_ref, acc_ref):
    j = pl.program_id(0)

    @pl.when(j == 0)
    def _():
        acc_ref[...] = jnp.zeros_like(acc_ref)

    a = agg_ref[0] + agg_ref[1] + hin_ref[...]
    h = jax.nn.relu(a * nd_ref[...] + b_ref[...])
    rows = lax.broadcasted_iota(jnp.int32, (BR, D), 0) + j * BR
    h = jnp.where(rows < N, h, 0.0)
    acc_ref[...] += jnp.sum(h, axis=0, keepdims=True)

    @pl.when(j == NPAD // BR - 1)
    def _():
        hg = acc_ref[...]
        a1 = jax.nn.relu(jnp.dot(hg, wf1_ref[...],
                                 preferred_element_type=jnp.float32,
                                 precision=_HIGH) + bf1_ref[...])
        a2 = jax.nn.relu(jnp.dot(a1, wf2_ref[...],
                                 preferred_element_type=jnp.float32,
                                 precision=_HIGH) + bf2_ref[...])
        o_ref[...] = jnp.dot(a2, wf3_ref[...],
                             preferred_element_type=jnp.float32,
                             precision=_HIGH) + bf3_ref[...]


def _final(agg, hin, ndst, b, Wf1, bf1, Wf2, bf2, Wf3p, bf3p):
    fc1, fc2 = Wf1.shape[1], Wf2.shape[1]
    return pl.pallas_call(
        _final_body,
        grid=(NPAD // BR,),
        in_specs=[
            pl.BlockSpec((NC, BR, D), lambda j: (0, j, 0)),
            pl.BlockSpec((BR, D), lambda j: (j, 0)),
            pl.BlockSpec((BR, 1), lambda j: (j, 0)),
            pl.BlockSpec((1, D), lambda j: (0, 0)),
            pl.BlockSpec((D, fc1), lambda j: (0, 0)),
            pl.BlockSpec((1, fc1), lambda j: (0, 0)),
            pl.BlockSpec((fc1, fc2), lambda j: (0, 0)),
            pl.BlockSpec((1, fc2), lambda j: (0, 0)),
            pl.BlockSpec((fc2, D), lambda j: (0, 0)),
            pl.BlockSpec((1, D), lambda j: (0, 0)),
        ],
        out_specs=pl.BlockSpec((1, D), lambda j: (0, 0)),
        out_shape=jax.ShapeDtypeStruct((1, D), jnp.float32),
        scratch_shapes=[pltpu.VMEM((1, D), jnp.float32)],
    )(agg, hin, ndst, b, Wf1, bf1, Wf2, bf2, Wf3p, bf3p)


_SPAD = NW * EWP - E
_PAD_IDS = N + (np.arange(_SPAD, dtype=np.int32) % (NPAD - N))


def kernel(x, edge_index, W1, b1, W2, b2, Wf1, bf1, Wf2, bf2, Wf3, bf3):
    src = edge_index[0]
    dst = edge_index[1]

    pad_ids = jnp.asarray(_PAD_IDS)
    sidx = jnp.concatenate([src, pad_ids]).reshape(NW, SNCH, SCK)
    didx = jnp.concatenate([dst, pad_ids]).reshape(NW, SNCH, SCK)

    norms = _degrees(sidx.reshape(NS, DNCH, DCK), didx.reshape(NS, DNCH, DCK))
    nsrc = norms[0].reshape(NPAD, 1)
    ndst = norms[1].reshape(NPAD, 1)

    xp = jnp.pad(x, ((0, NPAD - N), (0, 0)))
    b1r = b1.reshape(1, D)
    b2r = b2.reshape(1, D)
    bf1r = bf1.reshape(1, -1)
    bf2r = bf2.reshape(1, -1)
    Wf3p = jnp.pad(Wf3, ((0, 0), (0, D - Wf3.shape[1])))
    bf3p = jnp.pad(bf3, (0, D - bf3.shape[0])).reshape(1, D)

    h1in = _mm1(xp, nsrc, W1)
    agg1 = _edge_scatter(h1in, sidx, didx)
    h2in = _combine(agg1, h1in, nsrc, ndst, b1r, W2)
    agg2 = _edge_scatter(h2in, sidx, didx)
    ansp = _final(agg2, h2in, ndst, b2r, Wf1, bf1r, Wf2, bf2r, Wf3p, bf3p)
    return ansp[0, :Wf3.shape[1]]


# speedup vs baseline: 1.2482x; 1.2482x over previous
import functools

import jax
import jax.numpy as jnp
import numpy as np
from jax import lax
from jax.experimental import pallas as pl
from jax.experimental.pallas import tpu as pltpu
from jax.experimental.pallas import tpu_sc as plsc

N = 10000
E = 320000
D = 128
NPAD = 10240
NC, NS, L = 2, 16, 16
NW = NC * NS

SCK = 128
SNCH = 80
EWP = SCK * SNCH
DCK = 128
DNCH = 160
DEGPAD = 10496

_HIGH = jax.lax.Precision.DEFAULT


def _mesh():
    return plsc.VectorSubcoreMesh(
        core_axis_name="c", subcore_axis_name="s", num_cores=NC, num_subcores=NS
    )


def _fast_rsqrt16(v):
    i = lax.bitcast_convert_type(v, jnp.int32)
    i = jnp.int32(0x5F3759DF) - (i >> 1)
    y = lax.bitcast_convert_type(i, jnp.float32)
    for _ in range(3):
        y = y * (1.5 - 0.5 * v * y * y)
    return y


def _degree_body(sidx_hbm, didx_hbm, norms_hbm, idx_v, ones_v, init_v, buf_v,
                 sem, deg_sh):
    c = lax.axis_index("c")
    s = lax.axis_index("s")

    @pl.when(c == 0)
    def _():
        pltpu.sync_copy(sidx_hbm.at[s], idx_v)

    @pl.when(c == 1)
    def _():
        pltpu.sync_copy(didx_hbm.at[s], idx_v)

    def fill_ones(i, _):
        ones_v[pl.ds(i * L, L)] = jnp.full((L,), 1.0, jnp.float32)
        return _

    lax.fori_loop(0, DCK // L, fill_ones, None)

    def fill_init(i, _):
        init_v[pl.ds(i * L, L)] = jnp.full((L,), 1.0, jnp.float32)
        return _

    lax.fori_loop(0, (DEGPAD // NS) // L, fill_init, None)
    pltpu.sync_copy(init_v, deg_sh.at[pl.ds(s * (DEGPAD // NS), DEGPAD // NS)])
    plsc.subcore_barrier()

    def chunk(j, _):
        for k in range(8):
            pltpu.async_copy(ones_v, deg_sh.at[idx_v.at[j * 8 + k]], sem,
                             add=True)
        for k in range(8):
            pltpu.make_async_copy(
                ones_v, deg_sh.at[idx_v.at[j * 8 + k]], sem).wait()
        return _

    lax.fori_loop(0, DNCH // 8, chunk, None)
    plsc.subcore_barrier()

    nper = NPAD // NS
    pltpu.sync_copy(deg_sh.at[pl.ds(s * nper, nper)], buf_v)

    def rsq(i, _):
        buf_v[pl.ds(i * L, L)] = _fast_rsqrt16(buf_v[pl.ds(i * L, L)])
        return _

    lax.fori_loop(0, nper // L, rsq, None)
    pltpu.sync_copy(buf_v, norms_hbm.at[c, pl.ds(s * nper, nper)])


def _degrees(sidx_r, didx_r):
    kern = pl.kernel(
        _degree_body,
        out_type=jax.ShapeDtypeStruct((NC, NPAD), jnp.float32),
        mesh=_mesh(),
        scratch_types=[
            pltpu.VMEM((DNCH, DCK), jnp.int32),
            pltpu.VMEM((DCK,), jnp.float32),
            pltpu.VMEM((DEGPAD // NS,), jnp.float32),
            pltpu.VMEM((NPAD // NS,), jnp.float32),
            pltpu.SemaphoreType.DMA,
            pltpu.VMEM_SHARED((DEGPAD,), jnp.float32),
        ],
    )
    return kern(sidx_r, didx_r)


def _scatter_body(h_hbm, sidx_hbm, didx_hbm, out_hbm,
                  sidx_v, didx_v, rows_v, sem_a, sem_b, agg_sh):
    c = lax.axis_index("c")
    s = lax.axis_index("s")
    wid = c * NS + s
    nper = NPAD // NS

    def zrow(i, _):
        for k in range(D // L):
            rows_v[0, i, pl.ds(k * L, L)] = jnp.zeros((L,), jnp.float32)
        return _

    lax.fori_loop(0, SCK, zrow, None)
    for k in range(nper // SCK):
        pltpu.sync_copy(rows_v.at[0], agg_sh.at[pl.ds(s * nper + k * SCK, SCK)])

    plsc.subcore_barrier()

    rows0, rows1 = rows_v.at[0], rows_v.at[1]
    half = SNCH // 2
    for p in range(2):
        pltpu.sync_copy(sidx_hbm.at[wid, pl.ds(p * half, half)], sidx_v)
        pltpu.sync_copy(didx_hbm.at[wid, pl.ds(p * half, half)], didx_v)
        pltpu.async_copy(h_hbm.at[sidx_v.at[0]], rows0, sem_a)

        def body(i, _):
            g = 2 * i
            pltpu.async_copy(h_hbm.at[sidx_v.at[g + 1]], rows1, sem_b)
            pltpu.make_async_copy(h_hbm.at[sidx_v.at[g]], rows0, sem_a).wait()
            pltpu.sync_copy(rows0, agg_sh.at[didx_v.at[g]], add=True)

            @pl.when(i < half // 2 - 1)
            def _start_next():
                pltpu.async_copy(h_hbm.at[sidx_v.at[g + 2]], rows0, sem_a)

            pltpu.make_async_copy(h_hbm.at[sidx_v.at[g + 1]], rows1, sem_b).wait()
            pltpu.sync_copy(rows1, agg_sh.at[didx_v.at[g + 1]], add=True)
            return _

        lax.fori_loop(0, half // 2, body, None)
    plsc.subcore_barrier()

    pltpu.sync_copy(agg_sh.at[pl.ds(s * nper, nper)],
                    out_hbm.at[c, pl.ds(s * nper, nper)])


def _edge_scatter(h, sidx, didx):
    kern = pl.kernel(
        _scatter_body,
        out_type=jax.ShapeDtypeStruct((NC, NPAD, D), jnp.float32),
        mesh=_mesh(),
        scratch_types=[
            pltpu.VMEM((SNCH // 2, SCK), jnp.int32),
            pltpu.VMEM((SNCH // 2, SCK), jnp.int32),
            pltpu.VMEM((2, SCK, D), jnp.float32),
            pltpu.SemaphoreType.DMA,
            pltpu.SemaphoreType.DMA,
            pltpu.VMEM_SHARED((NPAD, D), jnp.float32),
        ],
    )
    return kern(h, sidx, didx)


BR = 2048


def _mm1_body(x_ref, ns_ref, w_ref, o_ref):
    o_ref[...] = jnp.dot(x_ref[...] * ns_ref[...], w_ref[...],
                         preferred_element_type=jnp.float32, precision=_HIGH)


def _mm1(xp, nsrc, W1):
    return pl.pallas_call(
        _mm1_body,
        grid=(NPAD // BR,),
        in_specs=[
            pl.BlockSpec((BR, D), lambda j: (j, 0)),
            pl.BlockSpec((BR, 1), lambda j: (j, 0)),
            pl.BlockSpec((D, D), lambda j: (0, 0)),
        ],
        out_specs=pl.BlockSpec((BR, D), lambda j: (j, 0)),
        out_shape=jax.ShapeDtypeStruct((NPAD, D), jnp.float32),
    )(xp, nsrc, W1)


def _combine_body(agg_ref, hin_ref, ns_ref, nd_ref, b_ref, w_ref, o_ref):
    a = agg_ref[0] + agg_ref[1] + hin_ref[...]
    h = jax.nn.relu(a * nd_ref[...] + b_ref[...])
    o_ref[...] = jnp.dot(h * ns_ref[...], w_ref[...],
                         preferred_element_type=jnp.float32, precision=_HIGH)


def _combine(agg, hin, nsrc, ndst, b, W):
    return pl.pallas_call(
        _combine_body,
        grid=(NPAD // BR,),
        in_specs=[
            pl.BlockSpec((NC, BR, D), lambda j: (0, j, 0)),
            pl.BlockSpec((BR, D), lambda j: (j, 0)),
            pl.BlockSpec((BR, 1), lambda j: (j, 0)),
            pl.BlockSpec((BR, 1), lambda j: (j, 0)),
            pl.BlockSpec((1, D), lambda j: (0, 0)),
            pl.BlockSpec((D, D), lambda j: (0, 0)),
        ],
        out_specs=pl.BlockSpec((BR, D), lambda j: (j, 0)),
        out_shape=jax.ShapeDtypeStruct((NPAD, D), jnp.float32),
    )(agg, hin, nsrc, ndst, b, W)


def _final_body(agg_ref, hin_ref, nd_ref, b_ref,
                wf1_ref, bf1_ref, wf2_ref, bf2_ref, wf3_ref, bf3_ref,
                o_ref, acc_ref):
    j = pl.program_id(0)

    @pl.when(j == 0)
    def _():
        acc_ref[...] = jnp.zeros_like(acc_ref)

    a = agg_ref[0] + agg_ref[1] + hin_ref[...]
    h = jax.nn.relu(a * nd_ref[...] + b_ref[...])
    rows = lax.broadcasted_iota(jnp.int32, (BR, D), 0) + j * BR
    h = jnp.where(rows < N, h, 0.0)
    acc_ref[...] += jnp.sum(h, axis=0, keepdims=True)

    @pl.when(j == NPAD // BR - 1)
    def _():
        hg = acc_ref[...]
        a1 = jax.nn.relu(jnp.dot(hg, wf1_ref[...],
                                 preferred_element_type=jnp.float32,
                                 precision=_HIGH) + bf1_ref[...])
        a2 = jax.nn.relu(jnp.dot(a1, wf2_ref[...],
                                 preferred_element_type=jnp.float32,
                                 precision=_HIGH) + bf2_ref[...])
        o_ref[...] = jnp.dot(a2, wf3_ref[...],
                             preferred_element_type=jnp.float32,
                             precision=_HIGH) + bf3_ref[...]


def _final(agg, hin, ndst, b, Wf1, bf1, Wf2, bf2, Wf3p, bf3p):
    fc1, fc2 = Wf1.shape[1], Wf2.shape[1]
    return pl.pallas_call(
        _final_body,
        grid=(NPAD // BR,),
        in_specs=[
            pl.BlockSpec((NC, BR, D), lambda j: (0, j, 0)),
            pl.BlockSpec((BR, D), lambda j: (j, 0)),
            pl.BlockSpec((BR, 1), lambda j: (j, 0)),
            pl.BlockSpec((1, D), lambda j: (0, 0)),
            pl.BlockSpec((D, fc1), lambda j: (0, 0)),
            pl.BlockSpec((1, fc1), lambda j: (0, 0)),
            pl.BlockSpec((fc1, fc2), lambda j: (0, 0)),
            pl.BlockSpec((1, fc2), lambda j: (0, 0)),
            pl.BlockSpec((fc2, D), lambda j: (0, 0)),
            pl.BlockSpec((1, D), lambda j: (0, 0)),
        ],
        out_specs=pl.BlockSpec((1, D), lambda j: (0, 0)),
        out_shape=jax.ShapeDtypeStruct((1, D), jnp.float32),
        scratch_shapes=[pltpu.VMEM((1, D), jnp.float32)],
    )(agg, hin, ndst, b, Wf1, bf1, Wf2, bf2, Wf3p, bf3p)


_SPAD = NW * EWP - E
_PAD_IDS = N + (np.arange(_SPAD, dtype=np.int32) % (NPAD - N))


def kernel(x, edge_index, W1, b1, W2, b2, Wf1, bf1, Wf2, bf2, Wf3, bf3):
    src = edge_index[0]
    dst = edge_index[1]

    pad_ids = jnp.asarray(_PAD_IDS)
    sidx = jnp.concatenate([src, pad_ids]).reshape(NW, SNCH, SCK)
    didx = jnp.concatenate([dst, pad_ids]).reshape(NW, SNCH, SCK)

    norms = _degrees(sidx.reshape(NS, DNCH, DCK), didx.reshape(NS, DNCH, DCK))
    nsrc = norms[0].reshape(NPAD, 1)
    ndst = norms[1].reshape(NPAD, 1)

    xp = jnp.pad(x, ((0, NPAD - N), (0, 0)))
    b1r = b1.reshape(1, D)
    b2r = b2.reshape(1, D)
    bf1r = bf1.reshape(1, -1)
    bf2r = bf2.reshape(1, -1)
    Wf3p = jnp.pad(Wf3, ((0, 0), (0, D - Wf3.shape[1])))
    bf3p = jnp.pad(bf3, (0, D - bf3.shape[0])).reshape(1, D)

    h1in = _mm1(xp, nsrc, W1)
    agg1 = _edge_scatter(h1in, sidx, didx)
    h2in = _combine(agg1, h1in, nsrc, ndst, b1r, W2)
    agg2 = _edge_scatter(h2in, sidx, didx)
    ansp = _final(agg2, h2in, ndst, b2r, Wf1, bf1r, Wf2, bf2r, Wf3p, bf3p)
    return ansp[0, :Wf3.shape[1]]


# speedup vs baseline: 1.2879x; 1.0318x over previous
import functools

import jax
import jax.numpy as jnp
import numpy as np
from jax import lax
from jax.experimental import pallas as pl
from jax.experimental.pallas import tpu as pltpu
from jax.experimental.pallas import tpu_sc as plsc

N = 10000
E = 320000
D = 128
NPAD = 10240
NC, NS, L = 2, 16, 16
NW = NC * NS

SCK = 128
SNCH = 80
EWP = SCK * SNCH
DCK = 128
DNCH = 160
DEGPAD = 10496

_HIGH = jax.lax.Precision.DEFAULT


def _mesh():
    return plsc.VectorSubcoreMesh(
        core_axis_name="c", subcore_axis_name="s", num_cores=NC, num_subcores=NS
    )


def _fast_rsqrt16(v):
    i = lax.bitcast_convert_type(v, jnp.int32)
    i = jnp.int32(0x5F3759DF) - (i >> 1)
    y = lax.bitcast_convert_type(i, jnp.float32)
    for _ in range(3):
        y = y * (1.5 - 0.5 * v * y * y)
    return y


def _degree_body(ei_hbm, norms_hbm, idx_v, ones_v, init_v, buf_v,
                 sem, deg_sh):
    c = lax.axis_index("c")
    s = lax.axis_index("s")

    pltpu.sync_copy(ei_hbm.at[c, s], idx_v)

    def fill_ones(i, _):
        ones_v[pl.ds(i * L, L)] = jnp.full((L,), 1.0, jnp.float32)
        return _

    lax.fori_loop(0, DCK // L, fill_ones, None)

    def fill_init(i, _):
        init_v[pl.ds(i * L, L)] = jnp.full((L,), 1.0, jnp.float32)
        return _

    lax.fori_loop(0, (DEGPAD // NS) // L, fill_init, None)
    pltpu.sync_copy(init_v, deg_sh.at[pl.ds(s * (DEGPAD // NS), DEGPAD // NS)])
    plsc.subcore_barrier()

    def chunk(j, _):
        for k in range(8):
            pltpu.async_copy(ones_v, deg_sh.at[idx_v.at[j * 8 + k]], sem,
                             add=True)
        for k in range(8):
            pltpu.make_async_copy(
                ones_v, deg_sh.at[idx_v.at[j * 8 + k]], sem).wait()
        return _

    lax.fori_loop(0, DNCH // 8, chunk, None)
    plsc.subcore_barrier()

    nper = NPAD // NS
    pltpu.sync_copy(deg_sh.at[pl.ds(s * nper, nper)], buf_v)

    def rsq(i, _):
        buf_v[pl.ds(i * L, L)] = _fast_rsqrt16(buf_v[pl.ds(i * L, L)])
        return _

    lax.fori_loop(0, nper // L, rsq, None)
    pltpu.sync_copy(buf_v, norms_hbm.at[c, pl.ds(s * nper, nper)])


def _degrees(ei_dg):
    kern = pl.kernel(
        _degree_body,
        out_type=jax.ShapeDtypeStruct((NC, NPAD), jnp.float32),
        mesh=_mesh(),
        scratch_types=[
            pltpu.VMEM((DNCH, DCK), jnp.int32),
            pltpu.VMEM((DCK,), jnp.float32),
            pltpu.VMEM((DEGPAD // NS,), jnp.float32),
            pltpu.VMEM((NPAD // NS,), jnp.float32),
            pltpu.SemaphoreType.DMA,
            pltpu.VMEM_SHARED((DEGPAD,), jnp.float32),
        ],
    )
    return kern(ei_dg)


def _scatter_body(h_hbm, ei_hbm, out_hbm,
                  sidx_v, didx_v, rows_v, sem_a, sem_b, agg_sh):
    c = lax.axis_index("c")
    s = lax.axis_index("s")
    wid = c * NS + s
    nper = NPAD // NS

    def zrow(i, _):
        for k in range(D // L):
            rows_v[0, i, pl.ds(k * L, L)] = jnp.zeros((L,), jnp.float32)
        return _

    lax.fori_loop(0, SCK, zrow, None)
    for k in range(nper // SCK):
        pltpu.sync_copy(rows_v.at[0], agg_sh.at[pl.ds(s * nper + k * SCK, SCK)])

    plsc.subcore_barrier()

    rows0, rows1 = rows_v.at[0], rows_v.at[1]
    half = SNCH // 2
    for p in range(2):
        pltpu.sync_copy(ei_hbm.at[0, wid, pl.ds(p * half, half)], sidx_v)
        pltpu.sync_copy(ei_hbm.at[1, wid, pl.ds(p * half, half)], didx_v)
        pltpu.async_copy(h_hbm.at[sidx_v.at[0]], rows0, sem_a)

        def body(i, _):
            g = 2 * i
            pltpu.async_copy(h_hbm.at[sidx_v.at[g + 1]], rows1, sem_b)
            pltpu.make_async_copy(h_hbm.at[sidx_v.at[g]], rows0, sem_a).wait()
            pltpu.sync_copy(rows0, agg_sh.at[didx_v.at[g]], add=True)

            @pl.when(i < half // 2 - 1)
            def _start_next():
                pltpu.async_copy(h_hbm.at[sidx_v.at[g + 2]], rows0, sem_a)

            pltpu.make_async_copy(h_hbm.at[sidx_v.at[g + 1]], rows1, sem_b).wait()
            pltpu.sync_copy(rows1, agg_sh.at[didx_v.at[g + 1]], add=True)
            return _

        lax.fori_loop(0, half // 2, body, None)
    plsc.subcore_barrier()

    pltpu.sync_copy(agg_sh.at[pl.ds(s * nper, nper)],
                    out_hbm.at[c, pl.ds(s * nper, nper)])


def _edge_scatter(h, ei_sc):
    kern = pl.kernel(
        _scatter_body,
        out_type=jax.ShapeDtypeStruct((NC, NPAD, D), jnp.float32),
        mesh=_mesh(),
        scratch_types=[
            pltpu.VMEM((SNCH // 2, SCK), jnp.int32),
            pltpu.VMEM((SNCH // 2, SCK), jnp.int32),
            pltpu.VMEM((2, SCK, D), jnp.float32),
            pltpu.SemaphoreType.DMA,
            pltpu.SemaphoreType.DMA,
            pltpu.VMEM_SHARED((NPAD, D), jnp.float32),
        ],
    )
    return kern(h, ei_sc)


BR = 2048


def _mm1_body(x_ref, ns_ref, w_ref, o_ref):
    o_ref[...] = jnp.dot(x_ref[...] * ns_ref[...], w_ref[...],
                         preferred_element_type=jnp.float32, precision=_HIGH)


def _mm1(xp, nsrc, W1):
    return pl.pallas_call(
        _mm1_body,
        grid=(NPAD // BR,),
        in_specs=[
            pl.BlockSpec((BR, D), lambda j: (j, 0)),
            pl.BlockSpec((BR, 1), lambda j: (j, 0)),
            pl.BlockSpec((D, D), lambda j: (0, 0)),
        ],
        out_specs=pl.BlockSpec((BR, D), lambda j: (j, 0)),
        out_shape=jax.ShapeDtypeStruct((NPAD, D), jnp.float32),
    )(xp, nsrc, W1)


def _combine_body(agg_ref, hin_ref, ns_ref, nd_ref, b_ref, w_ref, o_ref):
    a = agg_ref[0] + agg_ref[1] + hin_ref[...]
    h = jax.nn.relu(a * nd_ref[...] + b_ref[...])
    o_ref[...] = jnp.dot(h * ns_ref[...], w_ref[...],
                         preferred_element_type=jnp.float32, precision=_HIGH)


def _combine(agg, hin, nsrc, ndst, b, W):
    return pl.pallas_call(
        _combine_body,
        grid=(NPAD // BR,),
        in_specs=[
            pl.BlockSpec((NC, BR, D), lambda j: (0, j, 0)),
            pl.BlockSpec((BR, D), lambda j: (j, 0)),
            pl.BlockSpec((BR, 1), lambda j: (j, 0)),
            pl.BlockSpec((BR, 1), lambda j: (j, 0)),
            pl.BlockSpec((1, D), lambda j: (0, 0)),
            pl.BlockSpec((D, D), lambda j: (0, 0)),
        ],
        out_specs=pl.BlockSpec((BR, D), lambda j: (j, 0)),
        out_shape=jax.ShapeDtypeStruct((NPAD, D), jnp.float32),
    )(agg, hin, nsrc, ndst, b, W)


def _final_body(agg_ref, hin_ref, nd_ref, b_ref,
                wf1_ref, bf1_ref, wf2_ref, bf2_ref, wf3_ref, bf3_ref,
                o_ref, acc_ref):
    j = pl.program_id(0)

    @pl.when(j == 0)
    def _():
        acc_ref[...] = jnp.zeros_like(acc_ref)

    a = agg_ref[0] + agg_ref[1] + hin_ref[...]
    h = jax.nn.relu(a * nd_ref[...] + b_ref[...])
    rows = lax.broadcasted_iota(jnp.int32, (BR, D), 0) + j * BR
    h = jnp.where(rows < N, h, 0.0)
    acc_ref[...] += jnp.sum(h, axis=0, keepdims=True)

    @pl.when(j == NPAD // BR - 1)
    def _():
        hg = acc_ref[...]
        a1 = jax.nn.relu(jnp.dot(hg, wf1_ref[...],
                                 preferred_element_type=jnp.float32,
                                 precision=_HIGH) + bf1_ref[...])
        a2 = jax.nn.relu(jnp.dot(a1, wf2_ref[...],
                                 preferred_element_type=jnp.float32,
                                 precision=_HIGH) + bf2_ref[...])
        o_ref[...] = jnp.dot(a2, wf3_ref[...],
                             preferred_element_type=jnp.float32,
                             precision=_HIGH) + bf3_ref[...]


def _final(agg, hin, ndst, b, Wf1, bf1, Wf2, bf2, Wf3p, bf3p):
    fc1, fc2 = Wf1.shape[1], Wf2.shape[1]
    return pl.pallas_call(
        _final_body,
        grid=(NPAD // BR,),
        in_specs=[
            pl.BlockSpec((NC, BR, D), lambda j: (0, j, 0)),
            pl.BlockSpec((BR, D), lambda j: (j, 0)),
            pl.BlockSpec((BR, 1), lambda j: (j, 0)),
            pl.BlockSpec((1, D), lambda j: (0, 0)),
            pl.BlockSpec((D, fc1), lambda j: (0, 0)),
            pl.BlockSpec((1, fc1), lambda j: (0, 0)),
            pl.BlockSpec((fc1, fc2), lambda j: (0, 0)),
            pl.BlockSpec((1, fc2), lambda j: (0, 0)),
            pl.BlockSpec((fc2, D), lambda j: (0, 0)),
            pl.BlockSpec((1, D), lambda j: (0, 0)),
        ],
        out_specs=pl.BlockSpec((1, D), lambda j: (0, 0)),
        out_shape=jax.ShapeDtypeStruct((1, D), jnp.float32),
        scratch_shapes=[pltpu.VMEM((1, D), jnp.float32)],
    )(agg, hin, ndst, b, Wf1, bf1, Wf2, bf2, Wf3p, bf3p)


_SPAD = NW * EWP - E
_PAD_IDS = N + (np.arange(_SPAD, dtype=np.int32) % (NPAD - N))


def kernel(x, edge_index, W1, b1, W2, b2, Wf1, bf1, Wf2, bf2, Wf3, bf3):
    pad2 = jnp.asarray(np.stack([_PAD_IDS, _PAD_IDS]))
    ei_all = jnp.concatenate([edge_index, pad2], axis=1)
    ei_sc = ei_all.reshape(2, NW, SNCH, SCK)

    norms = _degrees(ei_all.reshape(2, NS, DNCH, DCK))
    nsrc = norms[0].reshape(NPAD, 1)
    ndst = norms[1].reshape(NPAD, 1)

    xp = jnp.pad(x, ((0, NPAD - N), (0, 0)))
    b1r = b1.reshape(1, D)
    b2r = b2.reshape(1, D)
    bf1r = bf1.reshape(1, -1)
    bf2r = bf2.reshape(1, -1)
    Wf3p = jnp.pad(Wf3, ((0, 0), (0, D - Wf3.shape[1])))
    bf3p = jnp.pad(bf3, (0, D - bf3.shape[0])).reshape(1, D)

    h1in = _mm1(xp, nsrc, W1)
    agg1 = _edge_scatter(h1in, ei_sc)
    h2in = _combine(agg1, h1in, nsrc, ndst, b1r, W2)
    agg2 = _edge_scatter(h2in, ei_sc)
    ansp = _final(agg2, h2in, ndst, b2r, Wf1, bf1r, Wf2, bf2r, Wf3p, bf3p)
    return ansp[0, :Wf3.shape[1]]


# speedup vs baseline: 1.2947x; 1.0053x over previous
import functools

import jax
import jax.numpy as jnp
import numpy as np
from jax import lax
from jax.experimental import pallas as pl
from jax.experimental.pallas import tpu as pltpu
from jax.experimental.pallas import tpu_sc as plsc

N = 10000
E = 320000
D = 128
NPAD = 10240
NC, NS, L = 2, 16, 16
NW = NC * NS

SCK = 128
SNCH = 80
EWP = SCK * SNCH
DCK = 128
DNCH = 160
DEGPAD = 10496

_HIGH = jax.lax.Precision.DEFAULT


def _mesh():
    return plsc.VectorSubcoreMesh(
        core_axis_name="c", subcore_axis_name="s", num_cores=NC, num_subcores=NS
    )


def _fast_rsqrt16(v):
    i = lax.bitcast_convert_type(v, jnp.int32)
    i = jnp.int32(0x5F3759DF) - (i >> 1)
    y = lax.bitcast_convert_type(i, jnp.float32)
    for _ in range(3):
        y = y * (1.5 - 0.5 * v * y * y)
    return y


def _degree_body(ei_hbm, norms_hbm, idx_v, ones_v, init_v, buf_v,
                 sem, deg_sh):
    c = lax.axis_index("c")
    s = lax.axis_index("s")

    pltpu.sync_copy(ei_hbm.at[c, s], idx_v)

    def fill_ones(i, _):
        ones_v[pl.ds(i * L, L)] = jnp.full((L,), 1.0, jnp.float32)
        return _

    lax.fori_loop(0, DCK // L, fill_ones, None)

    def fill_init(i, _):
        init_v[pl.ds(i * L, L)] = jnp.full((L,), 1.0, jnp.float32)
        return _

    lax.fori_loop(0, (DEGPAD // NS) // L, fill_init, None)
    pltpu.sync_copy(init_v, deg_sh.at[pl.ds(s * (DEGPAD // NS), DEGPAD // NS)])
    plsc.subcore_barrier()

    def chunk(j, _):
        for k in range(16):
            pltpu.async_copy(ones_v, deg_sh.at[idx_v.at[j * 16 + k]], sem,
                             add=True)
        for k in range(16):
            pltpu.make_async_copy(
                ones_v, deg_sh.at[idx_v.at[j * 16 + k]], sem).wait()
        return _

    lax.fori_loop(0, DNCH // 16, chunk, None)
    plsc.subcore_barrier()

    nper = NPAD // NS
    pltpu.sync_copy(deg_sh.at[pl.ds(s * nper, nper)], buf_v)

    def rsq(i, _):
        buf_v[pl.ds(i * L, L)] = _fast_rsqrt16(buf_v[pl.ds(i * L, L)])
        return _

    lax.fori_loop(0, nper // L, rsq, None)
    pltpu.sync_copy(buf_v, norms_hbm.at[c, pl.ds(s * nper, nper)])


def _degrees(ei_dg):
    kern = pl.kernel(
        _degree_body,
        out_type=jax.ShapeDtypeStruct((NC, NPAD), jnp.float32),
        mesh=_mesh(),
        scratch_types=[
            pltpu.VMEM((DNCH, DCK), jnp.int32),
            pltpu.VMEM((DCK,), jnp.float32),
            pltpu.VMEM((DEGPAD // NS,), jnp.float32),
            pltpu.VMEM((NPAD // NS,), jnp.float32),
            pltpu.SemaphoreType.DMA,
            pltpu.VMEM_SHARED((DEGPAD,), jnp.float32),
        ],
    )
    return kern(ei_dg)


def _scatter_body(h_hbm, ei_hbm, out_hbm,
                  sidx_v, didx_v, rows_v, sem_a, sem_b, agg_sh):
    c = lax.axis_index("c")
    s = lax.axis_index("s")
    wid = c * NS + s
    nper = NPAD // NS

    def zrow(i, _):
        for k in range(D // L):
            rows_v[0, i, pl.ds(k * L, L)] = jnp.zeros((L,), jnp.float32)
        return _

    lax.fori_loop(0, SCK, zrow, None)
    for k in range(nper // SCK):
        pltpu.sync_copy(rows_v.at[0], agg_sh.at[pl.ds(s * nper + k * SCK, SCK)])

    plsc.subcore_barrier()

    rows0, rows1 = rows_v.at[0], rows_v.at[1]
    half = SNCH // 2
    for p in range(2):
        pltpu.sync_copy(ei_hbm.at[0, wid, pl.ds(p * half, half)], sidx_v)
        pltpu.sync_copy(ei_hbm.at[1, wid, pl.ds(p * half, half)], didx_v)
        pltpu.async_copy(h_hbm.at[sidx_v.at[0]], rows0, sem_a)

        def body(i, _):
            g = 2 * i
            pltpu.async_copy(h_hbm.at[sidx_v.at[g + 1]], rows1, sem_b)
            pltpu.make_async_copy(h_hbm.at[sidx_v.at[g]], rows0, sem_a).wait()
            pltpu.sync_copy(rows0, agg_sh.at[didx_v.at[g]], add=True)

            @pl.when(i < half // 2 - 1)
            def _start_next():
                pltpu.async_copy(h_hbm.at[sidx_v.at[g + 2]], rows0, sem_a)

            pltpu.make_async_copy(h_hbm.at[sidx_v.at[g + 1]], rows1, sem_b).wait()
            pltpu.sync_copy(rows1, agg_sh.at[didx_v.at[g + 1]], add=True)
            return _

        lax.fori_loop(0, half // 2, body, None)
    plsc.subcore_barrier()

    pltpu.sync_copy(agg_sh.at[pl.ds(s * nper, nper)],
                    out_hbm.at[c, pl.ds(s * nper, nper)])


def _edge_scatter(h, ei_sc):
    kern = pl.kernel(
        _scatter_body,
        out_type=jax.ShapeDtypeStruct((NC, NPAD, D), jnp.float32),
        mesh=_mesh(),
        scratch_types=[
            pltpu.VMEM((SNCH // 2, SCK), jnp.int32),
            pltpu.VMEM((SNCH // 2, SCK), jnp.int32),
            pltpu.VMEM((2, SCK, D), jnp.float32),
            pltpu.SemaphoreType.DMA,
            pltpu.SemaphoreType.DMA,
            pltpu.VMEM_SHARED((NPAD, D), jnp.float32),
        ],
    )
    return kern(h, ei_sc)


BR = 2048


def _mm1_body(x_ref, ns_ref, w_ref, o_ref):
    o_ref[...] = jnp.dot(x_ref[...] * ns_ref[...], w_ref[...],
                         preferred_element_type=jnp.float32, precision=_HIGH)


def _mm1(xp, nsrc, W1):
    return pl.pallas_call(
        _mm1_body,
        grid=(NPAD // BR,),
        in_specs=[
            pl.BlockSpec((BR, D), lambda j: (j, 0)),
            pl.BlockSpec((BR, 1), lambda j: (j, 0)),
            pl.BlockSpec((D, D), lambda j: (0, 0)),
        ],
        out_specs=pl.BlockSpec((BR, D), lambda j: (j, 0)),
        out_shape=jax.ShapeDtypeStruct((NPAD, D), jnp.float32),
    )(xp, nsrc, W1)


def _combine_body(agg_ref, hin_ref, ns_ref, nd_ref, b_ref, w_ref, o_ref):
    a = agg_ref[0] + agg_ref[1] + hin_ref[...]
    h = jax.nn.relu(a * nd_ref[...] + b_ref[...])
    o_ref[...] = jnp.dot(h * ns_ref[...], w_ref[...],
                         preferred_element_type=jnp.float32, precision=_HIGH)


def _combine(agg, hin, nsrc, ndst, b, W):
    return pl.pallas_call(
        _combine_body,
        grid=(NPAD // BR,),
        in_specs=[
            pl.BlockSpec((NC, BR, D), lambda j: (0, j, 0)),
            pl.BlockSpec((BR, D), lambda j: (j, 0)),
            pl.BlockSpec((BR, 1), lambda j: (j, 0)),
            pl.BlockSpec((BR, 1), lambda j: (j, 0)),
            pl.BlockSpec((1, D), lambda j: (0, 0)),
            pl.BlockSpec((D, D), lambda j: (0, 0)),
        ],
        out_specs=pl.BlockSpec((BR, D), lambda j: (j, 0)),
        out_shape=jax.ShapeDtypeStruct((NPAD, D), jnp.float32),
    )(agg, hin, nsrc, ndst, b, W)


def _final_body(agg_ref, hin_ref, nd_ref, b_ref,
                wf1_ref, bf1_ref, wf2_ref, bf2_ref, wf3_ref, bf3_ref,
                o_ref, acc_ref):
    j = pl.program_id(0)

    @pl.when(j == 0)
    def _():
        acc_ref[...] = jnp.zeros_like(acc_ref)

    a = agg_ref[0] + agg_ref[1] + hin_ref[...]
    h = jax.nn.relu(a * nd_ref[...] + b_ref[...])
    rows = lax.broadcasted_iota(jnp.int32, (BR, D), 0) + j * BR
    h = jnp.where(rows < N, h, 0.0)
    acc_ref[...] += jnp.sum(h, axis=0, keepdims=True)

    @pl.when(j == NPAD // BR - 1)
    def _():
        hg = acc_ref[...]
        a1 = jax.nn.relu(jnp.dot(hg, wf1_ref[...],
                                 preferred_element_type=jnp.float32,
                                 precision=_HIGH) + bf1_ref[...])
        a2 = jax.nn.relu(jnp.dot(a1, wf2_ref[...],
                                 preferred_element_type=jnp.float32,
                                 precision=_HIGH) + bf2_ref[...])
        o_ref[...] = jnp.dot(a2, wf3_ref[...],
                             preferred_element_type=jnp.float32,
                             precision=_HIGH) + bf3_ref[...]


def _final(agg, hin, ndst, b, Wf1, bf1, Wf2, bf2, Wf3p, bf3p):
    fc1, fc2 = Wf1.shape[1], Wf2.shape[1]
    return pl.pallas_call(
        _final_body,
        grid=(NPAD // BR,),
        in_specs=[
            pl.BlockSpec((NC, BR, D), lambda j: (0, j, 0)),
            pl.BlockSpec((BR, D), lambda j: (j, 0)),
            pl.BlockSpec((BR, 1), lambda j: (j, 0)),
            pl.BlockSpec((1, D), lambda j: (0, 0)),
            pl.BlockSpec((D, fc1), lambda j: (0, 0)),
            pl.BlockSpec((1, fc1), lambda j: (0, 0)),
            pl.BlockSpec((fc1, fc2), lambda j: (0, 0)),
            pl.BlockSpec((1, fc2), lambda j: (0, 0)),
            pl.BlockSpec((fc2, D), lambda j: (0, 0)),
            pl.BlockSpec((1, D), lambda j: (0, 0)),
        ],
        out_specs=pl.BlockSpec((1, D), lambda j: (0, 0)),
        out_shape=jax.ShapeDtypeStruct((1, D), jnp.float32),
        scratch_shapes=[pltpu.VMEM((1, D), jnp.float32)],
    )(agg, hin, ndst, b, Wf1, bf1, Wf2, bf2, Wf3p, bf3p)


_SPAD = NW * EWP - E
_PAD_IDS = N + (np.arange(_SPAD, dtype=np.int32) % (NPAD - N))


def kernel(x, edge_index, W1, b1, W2, b2, Wf1, bf1, Wf2, bf2, Wf3, bf3):
    pad2 = jnp.asarray(np.stack([_PAD_IDS, _PAD_IDS]))
    ei_all = jnp.concatenate([edge_index, pad2], axis=1)
    ei_sc = ei_all.reshape(2, NW, SNCH, SCK)

    norms = _degrees(ei_all.reshape(2, NS, DNCH, DCK))
    nsrc = norms[0].reshape(NPAD, 1)
    ndst = norms[1].reshape(NPAD, 1)

    xp = jnp.pad(x, ((0, NPAD - N), (0, 0)))
    b1r = b1.reshape(1, D)
    b2r = b2.reshape(1, D)
    bf1r = bf1.reshape(1, -1)
    bf2r = bf2.reshape(1, -1)
    Wf3p = jnp.pad(Wf3, ((0, 0), (0, D - Wf3.shape[1])))
    bf3p = jnp.pad(bf3, (0, D - bf3.shape[0])).reshape(1, D)

    h1in = _mm1(xp, nsrc, W1)
    agg1 = _edge_scatter(h1in, ei_sc)
    h2in = _combine(agg1, h1in, nsrc, ndst, b1r, W2)
    agg2 = _edge_scatter(h2in, ei_sc)
    ansp = _final(agg2, h2in, ndst, b2r, Wf1, bf1r, Wf2, bf2r, Wf3p, bf3p)
    return ansp[0, :Wf3.shape[1]]
